# baseline (speedup 1.0000x reference)
import math

import jax
import jax.numpy as jnp
from jax import lax
from jax.experimental import pallas as pl
from jax.experimental.pallas import tpu as pltpu
from jax.experimental.pallas import tpu_sc as plsc

G = 512
NC = 2
NS = 16
CHUNK = 64
NBUF = 4
GD = 3
IB = 8
BN = 80


def _make_sc_agg(Dh, CH, Npad, trows, feat_split):
    RPT = Npad // NS
    NBLK = CH // IB
    assert NBLK % 2 == 0
    mesh = plsc.VectorSubcoreMesh(core_axis_name="c", subcore_axis_name="s")

    def body(table, srcg, dstg, zrow, agg_out,
             is0, is1, id0, id1, rows0, rows1, rows2, rows3, acc,
             isem0, isem1, rsem0, rsem1, rsem2, rsem3):
        c = lax.axis_index("c")
        s = lax.axis_index("s")
        isb = (is0, is1)
        idb = (id0, id1)
        rows = (rows0, rows1, rows2, rows3)
        isem = (isem0, isem1)
        rsem = (rsem0, rsem1, rsem2, rsem3)
        if feat_split:
            base = s * CH
            tview = table.at[pl.ds(c * trows, trows)]
        else:
            base = (c * NS + s) * CH
            tview = table

        def fetch_block(g, p):
            pltpu.async_copy(srcg.at[pl.ds(base + g * IB, IB)], isb[p],
                             isem[p])
            pltpu.async_copy(dstg.at[pl.ds(base + g * IB, IB)], idb[p],
                             isem[p])

        def wait_block(p):
            pltpu.make_async_copy(srcg.at[pl.ds(0, IB)], isb[p],
                                  isem[p]).wait()
            pltpu.make_async_copy(dstg.at[pl.ds(0, IB)], idb[p],
                                  isem[p]).wait()

        pltpu.sync_copy(zrow, acc.at[pl.ds(s * RPT, RPT)])
        fetch_block(0, 0)
        wait_block(0)
        for k in range(GD):
            pltpu.async_copy(tview.at[is0.at[k]], rows[k], rsem[k])
        plsc.subcore_barrier()

        def one_block(g, p):
            @pl.when(g + 1 < NBLK)
            def _():
                fetch_block(g + 1, 1 - p)

            for k in range(IB):
                ch = g * IB + k
                b = k % NBUF
                pltpu.make_async_copy(tview.at[isb[p].at[k]], rows[b],
                                      rsem[b]).wait()

                kk = k + GD
                nref = (isb[p].at[kk] if kk < IB
                        else isb[1 - p].at[kk - IB])
                nb = kk % NBUF

                @pl.when(ch + GD < CH)
                def _():
                    pltpu.async_copy(tview.at[nref], rows[nb], rsem[nb])

                pltpu.sync_copy(rows[b], acc.at[idb[p].at[k]], add=True)

                if k == IB - GD - 1:
                    @pl.when(g + 1 < NBLK)
                    def _():
                        wait_block(1 - p)

        def two_blocks(m, carry):
            one_block(m * 2, 0)
            one_block(m * 2 + 1, 1)
            return carry

        lax.fori_loop(0, NBLK // 2, two_blocks, 0)
        plsc.subcore_barrier()
        pltpu.sync_copy(acc.at[pl.ds(s * RPT, RPT)],
                        agg_out.at[c, pl.ds(s * RPT, RPT)])

    return pl.kernel(
        body,
        out_type=jax.ShapeDtypeStruct((NC, Npad, Dh), jnp.float32),
        mesh=mesh,
        scratch_types=[
            pltpu.VMEM((IB, CHUNK), jnp.int32),
            pltpu.VMEM((IB, CHUNK), jnp.int32),
            pltpu.VMEM((IB, CHUNK), jnp.int32),
            pltpu.VMEM((IB, CHUNK), jnp.int32),
            pltpu.VMEM((CHUNK, Dh), jnp.float32),
            pltpu.VMEM((CHUNK, Dh), jnp.float32),
            pltpu.VMEM((CHUNK, Dh), jnp.float32),
            pltpu.VMEM((CHUNK, Dh), jnp.float32),
            pltpu.VMEM_SHARED((Npad, Dh), jnp.float32),
            pltpu.SemaphoreType.DMA,
            pltpu.SemaphoreType.DMA,
            pltpu.SemaphoreType.DMA,
            pltpu.SemaphoreType.DMA,
            pltpu.SemaphoreType.DMA,
            pltpu.SemaphoreType.DMA,
        ],
    )


def _tc_layer1(N, D, TH, Hc, Npad):
    nb = N // BN

    def body(x_ref, alo, ahi, w, bvec, batch_r, bchunk, h_out, r_out):
        i = pl.program_id(0)
        hin = x_ref[...] + alo[0] + ahi[0]
        h = jnp.maximum(
            jnp.dot(hin, w[...], preferred_element_type=jnp.float32)
            + bvec[...], 0.0)
        h_out[0] = h[:, :D]
        h_out[1] = h[:, D:]
        bb = batch_r[0, 0, :]
        oh = (lax.broadcasted_iota(jnp.int32, (G, BN), 0)
              == bb[None, :]).astype(jnp.float32)

        @pl.when(i == 0)
        def _():
            r_out[...] = jnp.concatenate(
                [jnp.broadcast_to(bchunk[...], (G, bchunk.shape[1])),
                 jnp.zeros((G, TH - bchunk.shape[1]), jnp.float32)], axis=1)

        r_out[...] += jnp.dot(oh, h, preferred_element_type=jnp.float32)

    return pl.pallas_call(
        body,
        grid=(nb,),
        in_specs=[
            pl.BlockSpec((BN, D), lambda i: (i, 0)),
            pl.BlockSpec((1, BN, D), lambda i: (0, i, 0)),
            pl.BlockSpec((1, BN, D), lambda i: (1, i, 0)),
            pl.BlockSpec((D, TH), lambda i: (0, 0)),
            pl.BlockSpec((1, TH), lambda i: (0, 0)),
            pl.BlockSpec((1, 1, BN), lambda i: (i, 0, 0)),
            pl.BlockSpec((1, Hc), lambda i: (0, 0)),
        ],
        out_specs=[
            pl.BlockSpec((2, BN, D), lambda i: (0, i, 0)),
            pl.BlockSpec((G, TH), lambda i: (0, 0)),
        ],
        out_shape=[
            jax.ShapeDtypeStruct((2, N, D), jnp.float32),
            jax.ShapeDtypeStruct((G, TH), jnp.float32),
        ],
    )


def _tc_layer23(N, D, TH, Hc, Npad, write_h):
    nb = N // BN

    def body(hlo, hhi, alo, ahi, w, bvec, batch_r, bchunk, *outs):
        i = pl.program_id(0)
        if write_h:
            h_out, r_out = outs
        else:
            (r_out,) = outs
        hin = (jnp.concatenate([hlo[0], hhi[0]], axis=1)
               + jnp.concatenate([alo[0], ahi[0]], axis=1))
        h = jnp.maximum(
            jnp.dot(hin, w[...], preferred_element_type=jnp.float32)
            + bvec[...], 0.0)
        if write_h:
            h_out[0] = h[:, :D]
            h_out[1] = h[:, D:]
        bb = batch_r[0, 0, :]
        oh = (lax.broadcasted_iota(jnp.int32, (G, BN), 0)
              == bb[None, :]).astype(jnp.float32)

        @pl.when(i == 0)
        def _():
            r_out[...] = jnp.concatenate(
                [jnp.broadcast_to(bchunk[...], (G, bchunk.shape[1])),
                 jnp.zeros((G, TH - bchunk.shape[1]), jnp.float32)], axis=1)

        r_out[...] += jnp.dot(oh, h, preferred_element_type=jnp.float32)

    out_specs = [pl.BlockSpec((G, TH), lambda i: (0, 0))]
    out_shape = [jax.ShapeDtypeStruct((G, TH), jnp.float32)]
    if write_h:
        out_specs = [pl.BlockSpec((2, BN, D), lambda i: (0, i, 0))] + out_specs
        out_shape = [jax.ShapeDtypeStruct((2, N, D), jnp.float32)] + out_shape

    return pl.pallas_call(
        body,
        grid=(nb,),
        in_specs=[
            pl.BlockSpec((1, BN, D), lambda i: (0, i, 0)),
            pl.BlockSpec((1, BN, D), lambda i: (1, i, 0)),
            pl.BlockSpec((1, BN, D), lambda i: (0, i, 0)),
            pl.BlockSpec((1, BN, D), lambda i: (1, i, 0)),
            pl.BlockSpec((TH, TH), lambda i: (0, 0)),
            pl.BlockSpec((1, TH), lambda i: (0, 0)),
            pl.BlockSpec((1, 1, BN), lambda i: (i, 0, 0)),
            pl.BlockSpec((1, Hc), lambda i: (0, 0)),
        ],
        out_specs=out_specs,
        out_shape=out_shape,
    )


def kernel(x, edge_index, batch, W0, b0, W1, b1, W2, b2, bias):
    N, D = x.shape
    E = edge_index.shape[1]
    T, _, H = W0.shape
    TH = T * H
    L = 3

    CH1 = -(-E // (NC * NS * CHUNK))
    CH1 = -(-CH1 // (2 * IB)) * (2 * IB)
    Ep = NC * NS * CH1 * CHUNK
    CH2 = Ep // (NS * CHUNK)
    assert CH2 % (2 * IB) == 0
    align = (BN * NS * 8) // math.gcd(BN, NS * 8)
    Npad = -(-(N + 1) // align) * align
    assert N % BN == 0

    src = edge_index[0]
    dst = edge_index[1]
    pad = Ep - E
    srcp = jnp.concatenate([src, jnp.zeros((pad,), jnp.int32)])
    dstp = jnp.concatenate([dst, jnp.full((pad,), N, jnp.int32)])
    srcg = srcp.reshape(Ep // CHUNK, CHUNK)
    dstg = dstp.reshape(Ep // CHUNK, CHUNK)

    W0c = W0.transpose(1, 0, 2).reshape(D, TH)
    b0c = b0.reshape(1, TH)
    W1bd = jax.scipy.linalg.block_diag(*[W1[t] for t in range(T)])
    W2bd = jax.scipy.linalg.block_diag(*[W2[t] for t in range(T)])
    b1c = b1.reshape(1, TH)
    b2c = b2.reshape(1, TH)
    batch3 = batch.reshape(N // BN, 1, BN)
    bias_c = [bias[:, 0, l * H:(l + 1) * H] for l in range(L)]

    zrow = jnp.zeros((Npad // NS, D), jnp.float32)

    agg_l1 = _make_sc_agg(D, CH1, Npad, N, feat_split=False)
    agg_l23 = _make_sc_agg(D, CH2, Npad, N, feat_split=True)
    tc1 = _tc_layer1(N, D, TH, H, Npad)
    tc2 = _tc_layer23(N, D, TH, H, Npad, write_h=True)
    tc3 = _tc_layer23(N, D, TH, H, Npad, write_h=False)

    agg1 = agg_l1(x, srcg, dstg, zrow)
    h1, r1 = tc1(x, agg1, agg1, W0c, b0c, batch3, bias_c[0])
    agg2 = agg_l23(h1.reshape(2 * N, D), srcg, dstg, zrow)
    h2, r2 = tc2(h1, h1, agg2, agg2, W1bd, b1c, batch3, bias_c[1])
    agg3 = agg_l23(h2.reshape(2 * N, D), srcg, dstg, zrow)
    (r3,) = tc3(h2, h2, agg3, agg3, W2bd, b2c, batch3, bias_c[2])

    out = jnp.stack([r.reshape(G, T, H) for r in (r1, r2, r3)], axis=2)
    return out.reshape(G, T, L * H)


# speedup vs baseline: 1.0018x; 1.0018x over previous
import math

import jax
import jax.numpy as jnp
from jax import lax
from jax.experimental import pallas as pl
from jax.experimental.pallas import tpu as pltpu
from jax.experimental.pallas import tpu_sc as plsc

G = 512
NC = 2
NS = 16
CHUNK = 128
NBUF = 2
GD = 1
IB = 8
BN = 80


def _make_sc_agg(Dh, CH, Npad, trows, feat_split):
    RPT = Npad // NS
    NBLK = CH // IB
    assert NBLK % 2 == 0
    mesh = plsc.VectorSubcoreMesh(core_axis_name="c", subcore_axis_name="s")

    def body(table, srcg, dstg, zrow, agg_out,
             is0, is1, id0, id1, rows0, rows1, acc,
             isem0, isem1, rsem0, rsem1):
        c = lax.axis_index("c")
        s = lax.axis_index("s")
        isb = (is0, is1)
        idb = (id0, id1)
        rows = (rows0, rows1)
        isem = (isem0, isem1)
        rsem = (rsem0, rsem1)
        if feat_split:
            base = s * CH
            tview = table.at[pl.ds(c * trows, trows)]
        else:
            base = (c * NS + s) * CH
            tview = table

        def fetch_block(g, p):
            pltpu.async_copy(srcg.at[pl.ds(base + g * IB, IB)], isb[p],
                             isem[p])
            pltpu.async_copy(dstg.at[pl.ds(base + g * IB, IB)], idb[p],
                             isem[p])

        def wait_block(p):
            pltpu.make_async_copy(srcg.at[pl.ds(0, IB)], isb[p],
                                  isem[p]).wait()
            pltpu.make_async_copy(dstg.at[pl.ds(0, IB)], idb[p],
                                  isem[p]).wait()

        pltpu.sync_copy(zrow, acc.at[pl.ds(s * RPT, RPT)])
        fetch_block(0, 0)
        wait_block(0)
        for k in range(GD):
            pltpu.async_copy(tview.at[is0.at[k]], rows[k], rsem[k])
        plsc.subcore_barrier()

        def one_block(g, p):
            @pl.when(g + 1 < NBLK)
            def _():
                fetch_block(g + 1, 1 - p)

            for k in range(IB):
                ch = g * IB + k
                b = k % NBUF
                pltpu.make_async_copy(tview.at[isb[p].at[k]], rows[b],
                                      rsem[b]).wait()

                kk = k + GD
                nref = (isb[p].at[kk] if kk < IB
                        else isb[1 - p].at[kk - IB])
                nb = kk % NBUF

                @pl.when(ch + GD < CH)
                def _():
                    pltpu.async_copy(tview.at[nref], rows[nb], rsem[nb])

                pltpu.sync_copy(rows[b], acc.at[idb[p].at[k]], add=True)

                if k == IB - GD - 1:
                    @pl.when(g + 1 < NBLK)
                    def _():
                        wait_block(1 - p)

        def two_blocks(m, carry):
            one_block(m * 2, 0)
            one_block(m * 2 + 1, 1)
            return carry

        lax.fori_loop(0, NBLK // 2, two_blocks, 0)
        plsc.subcore_barrier()
        pltpu.sync_copy(acc.at[pl.ds(s * RPT, RPT)],
                        agg_out.at[c, pl.ds(s * RPT, RPT)])

    return pl.kernel(
        body,
        out_type=jax.ShapeDtypeStruct((NC, Npad, Dh), jnp.float32),
        mesh=mesh,
        scratch_types=[
            pltpu.VMEM((IB, CHUNK), jnp.int32),
            pltpu.VMEM((IB, CHUNK), jnp.int32),
            pltpu.VMEM((IB, CHUNK), jnp.int32),
            pltpu.VMEM((IB, CHUNK), jnp.int32),
            pltpu.VMEM((CHUNK, Dh), jnp.float32),
            pltpu.VMEM((CHUNK, Dh), jnp.float32),
            pltpu.VMEM_SHARED((Npad, Dh), jnp.float32),
            pltpu.SemaphoreType.DMA,
            pltpu.SemaphoreType.DMA,
            pltpu.SemaphoreType.DMA,
            pltpu.SemaphoreType.DMA,
        ],
    )


def _tc_layer1(N, D, TH, Hc, Npad):
    nb = N // BN

    def body(x_ref, alo, ahi, w, bvec, batch_r, bchunk, h_out, r_out):
        i = pl.program_id(0)
        hin = x_ref[...] + alo[0] + ahi[0]
        h = jnp.maximum(
            jnp.dot(hin, w[...], preferred_element_type=jnp.float32)
            + bvec[...], 0.0)
        h_out[0] = h[:, :D]
        h_out[1] = h[:, D:]
        bb = batch_r[0, 0, :]
        oh = (lax.broadcasted_iota(jnp.int32, (G, BN), 0)
              == bb[None, :]).astype(jnp.float32)

        @pl.when(i == 0)
        def _():
            r_out[...] = jnp.concatenate(
                [jnp.broadcast_to(bchunk[...], (G, bchunk.shape[1])),
                 jnp.zeros((G, TH - bchunk.shape[1]), jnp.float32)], axis=1)

        r_out[...] += jnp.dot(oh, h, preferred_element_type=jnp.float32)

    return pl.pallas_call(
        body,
        grid=(nb,),
        in_specs=[
            pl.BlockSpec((BN, D), lambda i: (i, 0)),
            pl.BlockSpec((1, BN, D), lambda i: (0, i, 0)),
            pl.BlockSpec((1, BN, D), lambda i: (1, i, 0)),
            pl.BlockSpec((D, TH), lambda i: (0, 0)),
            pl.BlockSpec((1, TH), lambda i: (0, 0)),
            pl.BlockSpec((1, 1, BN), lambda i: (i, 0, 0)),
            pl.BlockSpec((1, Hc), lambda i: (0, 0)),
        ],
        out_specs=[
            pl.BlockSpec((2, BN, D), lambda i: (0, i, 0)),
            pl.BlockSpec((G, TH), lambda i: (0, 0)),
        ],
        out_shape=[
            jax.ShapeDtypeStruct((2, N, D), jnp.float32),
            jax.ShapeDtypeStruct((G, TH), jnp.float32),
        ],
    )


def _tc_layer23(N, D, TH, Hc, Npad, write_h):
    nb = N // BN

    def body(hlo, hhi, alo, ahi, w, bvec, batch_r, bchunk, *outs):
        i = pl.program_id(0)
        if write_h:
            h_out, r_out = outs
        else:
            (r_out,) = outs
        hin = (jnp.concatenate([hlo[0], hhi[0]], axis=1)
               + jnp.concatenate([alo[0], ahi[0]], axis=1))
        h = jnp.maximum(
            jnp.dot(hin, w[...], preferred_element_type=jnp.float32)
            + bvec[...], 0.0)
        if write_h:
            h_out[0] = h[:, :D]
            h_out[1] = h[:, D:]
        bb = batch_r[0, 0, :]
        oh = (lax.broadcasted_iota(jnp.int32, (G, BN), 0)
              == bb[None, :]).astype(jnp.float32)

        @pl.when(i == 0)
        def _():
            r_out[...] = jnp.concatenate(
                [jnp.broadcast_to(bchunk[...], (G, bchunk.shape[1])),
                 jnp.zeros((G, TH - bchunk.shape[1]), jnp.float32)], axis=1)

        r_out[...] += jnp.dot(oh, h, preferred_element_type=jnp.float32)

    out_specs = [pl.BlockSpec((G, TH), lambda i: (0, 0))]
    out_shape = [jax.ShapeDtypeStruct((G, TH), jnp.float32)]
    if write_h:
        out_specs = [pl.BlockSpec((2, BN, D), lambda i: (0, i, 0))] + out_specs
        out_shape = [jax.ShapeDtypeStruct((2, N, D), jnp.float32)] + out_shape

    return pl.pallas_call(
        body,
        grid=(nb,),
        in_specs=[
            pl.BlockSpec((1, BN, D), lambda i: (0, i, 0)),
            pl.BlockSpec((1, BN, D), lambda i: (1, i, 0)),
            pl.BlockSpec((1, BN, D), lambda i: (0, i, 0)),
            pl.BlockSpec((1, BN, D), lambda i: (1, i, 0)),
            pl.BlockSpec((TH, TH), lambda i: (0, 0)),
            pl.BlockSpec((1, TH), lambda i: (0, 0)),
            pl.BlockSpec((1, 1, BN), lambda i: (i, 0, 0)),
            pl.BlockSpec((1, Hc), lambda i: (0, 0)),
        ],
        out_specs=out_specs,
        out_shape=out_shape,
    )


def kernel(x, edge_index, batch, W0, b0, W1, b1, W2, b2, bias):
    N, D = x.shape
    E = edge_index.shape[1]
    T, _, H = W0.shape
    TH = T * H
    L = 3

    CH1 = -(-E // (NC * NS * CHUNK))
    CH1 = -(-CH1 // (2 * IB)) * (2 * IB)
    Ep = NC * NS * CH1 * CHUNK
    CH2 = Ep // (NS * CHUNK)
    assert CH2 % (2 * IB) == 0
    align = (BN * NS * 8) // math.gcd(BN, NS * 8)
    Npad = -(-(N + 1) // align) * align
    assert N % BN == 0

    src = edge_index[0]
    dst = edge_index[1]
    pad = Ep - E
    srcp = jnp.concatenate([src, jnp.zeros((pad,), jnp.int32)])
    dstp = jnp.concatenate([dst, jnp.full((pad,), N, jnp.int32)])
    srcg = srcp.reshape(Ep // CHUNK, CHUNK)
    dstg = dstp.reshape(Ep // CHUNK, CHUNK)

    W0c = W0.transpose(1, 0, 2).reshape(D, TH)
    b0c = b0.reshape(1, TH)
    W1bd = jax.scipy.linalg.block_diag(*[W1[t] for t in range(T)])
    W2bd = jax.scipy.linalg.block_diag(*[W2[t] for t in range(T)])
    b1c = b1.reshape(1, TH)
    b2c = b2.reshape(1, TH)
    batch3 = batch.reshape(N // BN, 1, BN)
    bias_c = [bias[:, 0, l * H:(l + 1) * H] for l in range(L)]

    zrow = jnp.zeros((Npad // NS, D), jnp.float32)

    agg_l1 = _make_sc_agg(D, CH1, Npad, N, feat_split=False)
    agg_l23 = _make_sc_agg(D, CH2, Npad, N, feat_split=True)
    tc1 = _tc_layer1(N, D, TH, H, Npad)
    tc2 = _tc_layer23(N, D, TH, H, Npad, write_h=True)
    tc3 = _tc_layer23(N, D, TH, H, Npad, write_h=False)

    agg1 = agg_l1(x, srcg, dstg, zrow)
    h1, r1 = tc1(x, agg1, agg1, W0c, b0c, batch3, bias_c[0])
    agg2 = agg_l23(h1.reshape(2 * N, D), srcg, dstg, zrow)
    h2, r2 = tc2(h1, h1, agg2, agg2, W1bd, b1c, batch3, bias_c[1])
    agg3 = agg_l23(h2.reshape(2 * N, D), srcg, dstg, zrow)
    (r3,) = tc3(h2, h2, agg3, agg3, W2bd, b2c, batch3, bias_c[2])

    out = jnp.stack([r.reshape(G, T, H) for r in (r1, r2, r3)], axis=2)
    return out.reshape(G, T, L * H)


# speedup vs baseline: 1.0061x; 1.0043x over previous
import math

import jax
import jax.numpy as jnp
from jax import lax
from jax.experimental import pallas as pl
from jax.experimental.pallas import tpu as pltpu
from jax.experimental.pallas import tpu_sc as plsc

G = 512
NC = 2
NS = 16
CHUNK = 128
NBUF = 2
GD = 1
IB = 8
BN = 80


def _make_sc_agg(Dh, CH, Npad, trows, feat_split):
    RPT = Npad // NS
    NBLK = CH // IB
    EPR = NS * CH
    assert NBLK % 2 == 0
    mesh = plsc.VectorSubcoreMesh(core_axis_name="c", subcore_axis_name="s")

    def body(table, srcg, dstg, zrow, agg_out,
             is0, is1, id0, id1, rows0, rows1, acc,
             isem0, isem1, rsem0, rsem1):
        c = lax.axis_index("c")
        s = lax.axis_index("s")
        isb = (is0, is1)
        idb = (id0, id1)
        rows = (rows0, rows1)
        isem = (isem0, isem1)
        rsem = (rsem0, rsem1)
        if feat_split:
            base_s = c * EPR + s * CH
            base_d = s * CH
        else:
            base_s = (c * NS + s) * CH
            base_d = base_s
        tview = table

        def fetch_block(g, p):
            pltpu.async_copy(srcg.at[pl.ds(base_s + g * IB, IB)], isb[p],
                             isem[p])
            pltpu.async_copy(dstg.at[pl.ds(base_d + g * IB, IB)], idb[p],
                             isem[p])

        def wait_block(p):
            pltpu.make_async_copy(srcg.at[pl.ds(0, IB)], isb[p],
                                  isem[p]).wait()
            pltpu.make_async_copy(dstg.at[pl.ds(0, IB)], idb[p],
                                  isem[p]).wait()

        pltpu.sync_copy(zrow, acc.at[pl.ds(s * RPT, RPT)])
        fetch_block(0, 0)
        wait_block(0)
        for k in range(GD):
            pltpu.async_copy(tview.at[is0.at[k]], rows[k], rsem[k])
        plsc.subcore_barrier()

        def one_block(g, p):
            @pl.when(g + 1 < NBLK)
            def _():
                fetch_block(g + 1, 1 - p)

            for k in range(IB):
                ch = g * IB + k
                b = k % NBUF
                pltpu.make_async_copy(tview.at[isb[p].at[k]], rows[b],
                                      rsem[b]).wait()

                kk = k + GD
                nref = (isb[p].at[kk] if kk < IB
                        else isb[1 - p].at[kk - IB])
                nb = kk % NBUF

                @pl.when(ch + GD < CH)
                def _():
                    pltpu.async_copy(tview.at[nref], rows[nb], rsem[nb])

                pltpu.sync_copy(rows[b], acc.at[idb[p].at[k]], add=True)

                if k == IB - GD - 1:
                    @pl.when(g + 1 < NBLK)
                    def _():
                        wait_block(1 - p)

        def two_blocks(m, carry):
            one_block(m * 2, 0)
            one_block(m * 2 + 1, 1)
            return carry

        lax.fori_loop(0, NBLK // 2, two_blocks, 0)
        plsc.subcore_barrier()
        pltpu.sync_copy(acc.at[pl.ds(s * RPT, RPT)],
                        agg_out.at[c, pl.ds(s * RPT, RPT)])

    return pl.kernel(
        body,
        out_type=jax.ShapeDtypeStruct((NC, Npad, Dh), jnp.float32),
        mesh=mesh,
        scratch_types=[
            pltpu.VMEM((IB, CHUNK), jnp.int32),
            pltpu.VMEM((IB, CHUNK), jnp.int32),
            pltpu.VMEM((IB, CHUNK), jnp.int32),
            pltpu.VMEM((IB, CHUNK), jnp.int32),
            pltpu.VMEM((CHUNK, Dh), jnp.float32),
            pltpu.VMEM((CHUNK, Dh), jnp.float32),
            pltpu.VMEM_SHARED((Npad, Dh), jnp.float32),
            pltpu.SemaphoreType.DMA,
            pltpu.SemaphoreType.DMA,
            pltpu.SemaphoreType.DMA,
            pltpu.SemaphoreType.DMA,
        ],
    )


def _tc_layer1(N, D, TH, Hc, Npad):
    nb = N // BN

    def body(x_ref, alo, ahi, w, bvec, batch_r, bchunk, h_out, r_out):
        i = pl.program_id(0)
        hin = x_ref[...] + alo[0] + ahi[0]
        h = jnp.maximum(
            jnp.dot(hin, w[...], preferred_element_type=jnp.float32)
            + bvec[...], 0.0)
        h_out[0] = h[:, :D]
        h_out[1] = h[:, D:]
        bb = batch_r[0, 0, :]
        oh = (lax.broadcasted_iota(jnp.int32, (G, BN), 0)
              == bb[None, :]).astype(jnp.float32)

        @pl.when(i == 0)
        def _():
            r_out[...] = jnp.concatenate(
                [jnp.broadcast_to(bchunk[...], (G, bchunk.shape[1])),
                 jnp.zeros((G, TH - bchunk.shape[1]), jnp.float32)], axis=1)

        r_out[...] += jnp.dot(oh, h, preferred_element_type=jnp.float32)

    return pl.pallas_call(
        body,
        grid=(nb,),
        in_specs=[
            pl.BlockSpec((BN, D), lambda i: (i, 0)),
            pl.BlockSpec((1, BN, D), lambda i: (0, i, 0)),
            pl.BlockSpec((1, BN, D), lambda i: (1, i, 0)),
            pl.BlockSpec((D, TH), lambda i: (0, 0)),
            pl.BlockSpec((1, TH), lambda i: (0, 0)),
            pl.BlockSpec((1, 1, BN), lambda i: (i, 0, 0)),
            pl.BlockSpec((1, Hc), lambda i: (0, 0)),
        ],
        out_specs=[
            pl.BlockSpec((2, BN, D), lambda i: (0, i, 0)),
            pl.BlockSpec((G, TH), lambda i: (0, 0)),
        ],
        out_shape=[
            jax.ShapeDtypeStruct((2, N, D), jnp.float32),
            jax.ShapeDtypeStruct((G, TH), jnp.float32),
        ],
    )


def _tc_layer23(N, D, TH, Hc, Npad, write_h):
    nb = N // BN

    def body(hlo, hhi, alo, ahi, w, bvec, batch_r, bchunk, *outs):
        i = pl.program_id(0)
        if write_h:
            h_out, r_out = outs
        else:
            (r_out,) = outs
        hin = (jnp.concatenate([hlo[0], hhi[0]], axis=1)
               + jnp.concatenate([alo[0], ahi[0]], axis=1))
        h = jnp.maximum(
            jnp.dot(hin, w[...], preferred_element_type=jnp.float32)
            + bvec[...], 0.0)
        if write_h:
            h_out[0] = h[:, :D]
            h_out[1] = h[:, D:]
        bb = batch_r[0, 0, :]
        oh = (lax.broadcasted_iota(jnp.int32, (G, BN), 0)
              == bb[None, :]).astype(jnp.float32)

        @pl.when(i == 0)
        def _():
            r_out[...] = jnp.concatenate(
                [jnp.broadcast_to(bchunk[...], (G, bchunk.shape[1])),
                 jnp.zeros((G, TH - bchunk.shape[1]), jnp.float32)], axis=1)

        r_out[...] += jnp.dot(oh, h, preferred_element_type=jnp.float32)

    out_specs = [pl.BlockSpec((G, TH), lambda i: (0, 0))]
    out_shape = [jax.ShapeDtypeStruct((G, TH), jnp.float32)]
    if write_h:
        out_specs = [pl.BlockSpec((2, BN, D), lambda i: (0, i, 0))] + out_specs
        out_shape = [jax.ShapeDtypeStruct((2, N, D), jnp.float32)] + out_shape

    return pl.pallas_call(
        body,
        grid=(nb,),
        in_specs=[
            pl.BlockSpec((1, BN, D), lambda i: (0, i, 0)),
            pl.BlockSpec((1, BN, D), lambda i: (1, i, 0)),
            pl.BlockSpec((1, BN, D), lambda i: (0, i, 0)),
            pl.BlockSpec((1, BN, D), lambda i: (1, i, 0)),
            pl.BlockSpec((TH, TH), lambda i: (0, 0)),
            pl.BlockSpec((1, TH), lambda i: (0, 0)),
            pl.BlockSpec((1, 1, BN), lambda i: (i, 0, 0)),
            pl.BlockSpec((1, Hc), lambda i: (0, 0)),
        ],
        out_specs=out_specs,
        out_shape=out_shape,
    )


def kernel(x, edge_index, batch, W0, b0, W1, b1, W2, b2, bias):
    N, D = x.shape
    E = edge_index.shape[1]
    T, _, H = W0.shape
    TH = T * H
    L = 3

    CH1 = -(-E // (NC * NS * CHUNK))
    CH1 = -(-CH1 // (2 * IB)) * (2 * IB)
    Ep = NC * NS * CH1 * CHUNK
    CH2 = Ep // (NS * CHUNK)
    assert CH2 % (2 * IB) == 0
    align = (BN * NS * 8) // math.gcd(BN, NS * 8)
    Npad = -(-(N + 1) // align) * align
    assert N % BN == 0

    src = edge_index[0]
    dst = edge_index[1]
    pad = Ep - E
    srcp = jnp.concatenate([src, jnp.zeros((pad,), jnp.int32)])
    dstp = jnp.concatenate([dst, jnp.full((pad,), N, jnp.int32)])
    srcg = srcp.reshape(Ep // CHUNK, CHUNK)
    dstg = dstp.reshape(Ep // CHUNK, CHUNK)
    src2g = jnp.concatenate([srcp, srcp + N]).reshape(2 * Ep // CHUNK, CHUNK)

    W0c = W0.transpose(1, 0, 2).reshape(D, TH)
    b0c = b0.reshape(1, TH)
    W1bd = jax.scipy.linalg.block_diag(*[W1[t] for t in range(T)])
    W2bd = jax.scipy.linalg.block_diag(*[W2[t] for t in range(T)])
    b1c = b1.reshape(1, TH)
    b2c = b2.reshape(1, TH)
    batch3 = batch.reshape(N // BN, 1, BN)
    bias_c = [bias[:, 0, l * H:(l + 1) * H] for l in range(L)]

    zrow = jnp.zeros((Npad // NS, D), jnp.float32)

    agg_l1 = _make_sc_agg(D, CH1, Npad, N, feat_split=False)
    agg_l23 = _make_sc_agg(D, CH2, Npad, N, feat_split=True)
    tc1 = _tc_layer1(N, D, TH, H, Npad)
    tc2 = _tc_layer23(N, D, TH, H, Npad, write_h=True)
    tc3 = _tc_layer23(N, D, TH, H, Npad, write_h=False)

    agg1 = agg_l1(x, srcg, dstg, zrow)
    h1, r1 = tc1(x, agg1, agg1, W0c, b0c, batch3, bias_c[0])
    agg2 = agg_l23(h1.reshape(2 * N, D), src2g, dstg, zrow)
    h2, r2 = tc2(h1, h1, agg2, agg2, W1bd, b1c, batch3, bias_c[1])
    agg3 = agg_l23(h2.reshape(2 * N, D), src2g, dstg, zrow)
    (r3,) = tc3(h2, h2, agg3, agg3, W2bd, b2c, batch3, bias_c[2])

    out = jnp.stack([r.reshape(G, T, H) for r in (r1, r2, r3)], axis=2)
    return out.reshape(G, T, L * H)


# speedup vs baseline: 1.0407x; 1.0344x over previous
import math

import jax
import jax.numpy as jnp
from jax import lax
from jax.experimental import pallas as pl
from jax.experimental.pallas import tpu as pltpu
from jax.experimental.pallas import tpu_sc as plsc

G = 512
NC = 2
NS = 16
CHUNK = 128
NBUF = 2
GD = 1
IB = 8
BN = 80


def _make_sc_agg(Dh, CH, Npad, trows, feat_split):
    RPT = Npad // NS
    NBLK = CH // IB
    EPR = NS * CH
    assert NBLK % 2 == 0
    mesh = plsc.VectorSubcoreMesh(core_axis_name="c", subcore_axis_name="s")

    def body(table, srcg, dstg, zrow, agg_out,
             is0, is1, id0, id1, rows0, rows1, acc,
             isem0, isem1, rsem0, rsem1):
        c = lax.axis_index("c")
        s = lax.axis_index("s")
        isb = (is0, is1)
        idb = (id0, id1)
        rows = (rows0, rows1)
        isem = (isem0, isem1)
        rsem = (rsem0, rsem1)
        if feat_split:
            base_s = c * EPR + s * CH
            base_d = s * CH
        else:
            base_s = (c * NS + s) * CH
            base_d = base_s
        tview = table

        def fetch_idx(ch, p):
            pltpu.async_copy(srcg.at[pl.ds(base_s + ch, 1)], isb[p],
                             isem[p])
            pltpu.async_copy(dstg.at[pl.ds(base_d + ch, 1)], idb[p],
                             isem[p])

        def wait_idx(p):
            pltpu.make_async_copy(srcg.at[pl.ds(0, 1)], isb[p],
                                  isem[p]).wait()
            pltpu.make_async_copy(dstg.at[pl.ds(0, 1)], idb[p],
                                  isem[p]).wait()

        pltpu.sync_copy(zrow, acc.at[pl.ds(s * RPT, RPT)])
        fetch_idx(0, 0)
        fetch_idx(1, 1)
        wait_idx(0)
        pltpu.async_copy(tview.at[is0.at[0]], rows[0], rsem[0])
        plsc.subcore_barrier()

        def step(ch, b):
            nb = 1 - b

            @pl.when(ch + 1 < CH)
            def _():
                wait_idx(nb)
                pltpu.async_copy(tview.at[isb[nb].at[0]], rows[nb],
                                 rsem[nb])

            pltpu.make_async_copy(tview.at[isb[b].at[0]], rows[b],
                                  rsem[b]).wait()
            pltpu.sync_copy(rows[b], acc.at[idb[b].at[0]], add=True)

            @pl.when(ch + 2 < CH)
            def _():
                fetch_idx(ch + 2, b)

        def two_steps(m, carry):
            step(m * 2, 0)
            step(m * 2 + 1, 1)
            return carry

        lax.fori_loop(0, CH // 2, two_steps, 0)
        plsc.subcore_barrier()
        pltpu.sync_copy(acc.at[pl.ds(s * RPT, RPT)],
                        agg_out.at[c, pl.ds(s * RPT, RPT)])

    return pl.kernel(
        body,
        out_type=jax.ShapeDtypeStruct((NC, Npad, Dh), jnp.float32),
        mesh=mesh,
        scratch_types=[
            pltpu.VMEM((1, CHUNK), jnp.int32),
            pltpu.VMEM((1, CHUNK), jnp.int32),
            pltpu.VMEM((1, CHUNK), jnp.int32),
            pltpu.VMEM((1, CHUNK), jnp.int32),
            pltpu.VMEM((CHUNK, Dh), jnp.float32),
            pltpu.VMEM((CHUNK, Dh), jnp.float32),
            pltpu.VMEM_SHARED((Npad, Dh), jnp.float32),
            pltpu.SemaphoreType.DMA,
            pltpu.SemaphoreType.DMA,
            pltpu.SemaphoreType.DMA,
            pltpu.SemaphoreType.DMA,
        ],
    )


def _tc_layer1(N, D, TH, Hc, Npad):
    nb = N // BN

    def body(x_ref, alo, ahi, w, bvec, batch_r, bchunk, h_out, r_out):
        i = pl.program_id(0)
        hin = x_ref[...] + alo[0] + ahi[0]
        h = jnp.maximum(
            jnp.dot(hin, w[...], preferred_element_type=jnp.float32)
            + bvec[...], 0.0)
        h_out[0] = h[:, :D]
        h_out[1] = h[:, D:]
        bb = batch_r[0, 0, :]
        oh = (lax.broadcasted_iota(jnp.int32, (G, BN), 0)
              == bb[None, :]).astype(jnp.float32)

        @pl.when(i == 0)
        def _():
            r_out[...] = jnp.concatenate(
                [jnp.broadcast_to(bchunk[...], (G, bchunk.shape[1])),
                 jnp.zeros((G, TH - bchunk.shape[1]), jnp.float32)], axis=1)

        r_out[...] += jnp.dot(oh, h, preferred_element_type=jnp.float32)

    return pl.pallas_call(
        body,
        grid=(nb,),
        in_specs=[
            pl.BlockSpec((BN, D), lambda i: (i, 0)),
            pl.BlockSpec((1, BN, D), lambda i: (0, i, 0)),
            pl.BlockSpec((1, BN, D), lambda i: (1, i, 0)),
            pl.BlockSpec((D, TH), lambda i: (0, 0)),
            pl.BlockSpec((1, TH), lambda i: (0, 0)),
            pl.BlockSpec((1, 1, BN), lambda i: (i, 0, 0)),
            pl.BlockSpec((1, Hc), lambda i: (0, 0)),
        ],
        out_specs=[
            pl.BlockSpec((2, BN, D), lambda i: (0, i, 0)),
            pl.BlockSpec((G, TH), lambda i: (0, 0)),
        ],
        out_shape=[
            jax.ShapeDtypeStruct((2, N, D), jnp.float32),
            jax.ShapeDtypeStruct((G, TH), jnp.float32),
        ],
    )


def _tc_layer23(N, D, TH, Hc, Npad, write_h):
    nb = N // BN

    def body(hlo, hhi, alo, ahi, w, bvec, batch_r, bchunk, *outs):
        i = pl.program_id(0)
        if write_h:
            h_out, r_out = outs
        else:
            (r_out,) = outs
        hin = (jnp.concatenate([hlo[0], hhi[0]], axis=1)
               + jnp.concatenate([alo[0], ahi[0]], axis=1))
        h = jnp.maximum(
            jnp.dot(hin, w[...], preferred_element_type=jnp.float32)
            + bvec[...], 0.0)
        if write_h:
            h_out[0] = h[:, :D]
            h_out[1] = h[:, D:]
        bb = batch_r[0, 0, :]
        oh = (lax.broadcasted_iota(jnp.int32, (G, BN), 0)
              == bb[None, :]).astype(jnp.float32)

        @pl.when(i == 0)
        def _():
            r_out[...] = jnp.concatenate(
                [jnp.broadcast_to(bchunk[...], (G, bchunk.shape[1])),
                 jnp.zeros((G, TH - bchunk.shape[1]), jnp.float32)], axis=1)

        r_out[...] += jnp.dot(oh, h, preferred_element_type=jnp.float32)

    out_specs = [pl.BlockSpec((G, TH), lambda i: (0, 0))]
    out_shape = [jax.ShapeDtypeStruct((G, TH), jnp.float32)]
    if write_h:
        out_specs = [pl.BlockSpec((2, BN, D), lambda i: (0, i, 0))] + out_specs
        out_shape = [jax.ShapeDtypeStruct((2, N, D), jnp.float32)] + out_shape

    return pl.pallas_call(
        body,
        grid=(nb,),
        in_specs=[
            pl.BlockSpec((1, BN, D), lambda i: (0, i, 0)),
            pl.BlockSpec((1, BN, D), lambda i: (1, i, 0)),
            pl.BlockSpec((1, BN, D), lambda i: (0, i, 0)),
            pl.BlockSpec((1, BN, D), lambda i: (1, i, 0)),
            pl.BlockSpec((TH, TH), lambda i: (0, 0)),
            pl.BlockSpec((1, TH), lambda i: (0, 0)),
            pl.BlockSpec((1, 1, BN), lambda i: (i, 0, 0)),
            pl.BlockSpec((1, Hc), lambda i: (0, 0)),
        ],
        out_specs=out_specs,
        out_shape=out_shape,
    )


def kernel(x, edge_index, batch, W0, b0, W1, b1, W2, b2, bias):
    N, D = x.shape
    E = edge_index.shape[1]
    T, _, H = W0.shape
    TH = T * H
    L = 3

    CH1 = -(-E // (NC * NS * CHUNK))
    CH1 = -(-CH1 // (2 * IB)) * (2 * IB)
    Ep = NC * NS * CH1 * CHUNK
    CH2 = Ep // (NS * CHUNK)
    assert CH2 % (2 * IB) == 0
    align = (BN * NS * 8) // math.gcd(BN, NS * 8)
    Npad = -(-(N + 1) // align) * align
    assert N % BN == 0

    src = edge_index[0]
    dst = edge_index[1]
    pad = Ep - E
    srcp = jnp.concatenate([src, jnp.zeros((pad,), jnp.int32)])
    dstp = jnp.concatenate([dst, jnp.full((pad,), N, jnp.int32)])
    srcg = srcp.reshape(Ep // CHUNK, CHUNK)
    dstg = dstp.reshape(Ep // CHUNK, CHUNK)
    src2g = jnp.concatenate([srcp, srcp + N]).reshape(2 * Ep // CHUNK, CHUNK)

    W0c = W0.transpose(1, 0, 2).reshape(D, TH)
    b0c = b0.reshape(1, TH)
    W1bd = jax.scipy.linalg.block_diag(*[W1[t] for t in range(T)])
    W2bd = jax.scipy.linalg.block_diag(*[W2[t] for t in range(T)])
    b1c = b1.reshape(1, TH)
    b2c = b2.reshape(1, TH)
    batch3 = batch.reshape(N // BN, 1, BN)
    bias_c = [bias[:, 0, l * H:(l + 1) * H] for l in range(L)]

    zrow = jnp.zeros((Npad // NS, D), jnp.float32)

    agg_l1 = _make_sc_agg(D, CH1, Npad, N, feat_split=False)
    agg_l23 = _make_sc_agg(D, CH2, Npad, N, feat_split=True)
    tc1 = _tc_layer1(N, D, TH, H, Npad)
    tc2 = _tc_layer23(N, D, TH, H, Npad, write_h=True)
    tc3 = _tc_layer23(N, D, TH, H, Npad, write_h=False)

    agg1 = agg_l1(x, srcg, dstg, zrow)
    h1, r1 = tc1(x, agg1, agg1, W0c, b0c, batch3, bias_c[0])
    agg2 = agg_l23(h1.reshape(2 * N, D), src2g, dstg, zrow)
    h2, r2 = tc2(h1, h1, agg2, agg2, W1bd, b1c, batch3, bias_c[1])
    agg3 = agg_l23(h2.reshape(2 * N, D), src2g, dstg, zrow)
    (r3,) = tc3(h2, h2, agg3, agg3, W2bd, b2c, batch3, bias_c[2])

    out = jnp.stack([r.reshape(G, T, H) for r in (r1, r2, r3)], axis=2)
    return out.reshape(G, T, L * H)
